# Initial kernel scaffold; baseline (speedup 1.0000x reference)
import jax
import jax.numpy as jnp
from jax.experimental import pallas as pl


def kernel(output, tag_pull, tag_push, mask_pull, mask_push):
    raise NotImplementedError("write your pallas kernel here")


# speedup vs baseline: 1.5531x; 1.5531x over previous
import functools

import jax
import jax.numpy as jnp
from jax import lax
from jax.experimental import pallas as pl
from jax.experimental.pallas import tpu as pltpu
from jax.experimental.pallas import tpu_sc as plsc

B, C, H, W, K = 32, 4, 128, 128, 128
HW = H * W
L = 16


def _rsqrt_plus_eps_inv(s):
    xi = plsc.bitcast(s, jnp.int32)
    yi = jnp.int32(0x5F3759DF) - lax.shift_right_logical(xi, 1)
    y = plsc.bitcast(yi, jnp.float32)
    for _ in range(3):
        y = y * (1.5 - 0.5 * s * y * y)
    sqrt_s = s * y
    return 1.0 / (sqrt_s + 1e-10)


def _sc_partials(feat, ip0, ip1, iq0, iq1, mp, mq):
    info = plsc.get_sparse_core_info()
    nc = info.num_cores
    mesh = plsc.VectorSubcoreMesh(core_axis_name="c", subcore_axis_name="s")

    @functools.partial(
        pl.kernel,
        mesh=mesh,
        out_type=jax.ShapeDtypeStruct((4 * B, L), jnp.float32),
        compiler_params=pltpu.CompilerParams(needs_layout_passes=False),
        scratch_types=[
            pltpu.VMEM((C * HW,), jnp.float32),
            pltpu.VMEM((K,), jnp.int32),
            pltpu.VMEM((K,), jnp.int32),
            pltpu.VMEM((K,), jnp.int32),
            pltpu.VMEM((K,), jnp.int32),
            pltpu.VMEM((K,), jnp.float32),
            pltpu.VMEM((K,), jnp.float32),
            pltpu.VMEM((L,), jnp.float32),
        ],
    )
    def body(feat_hbm, ip0_hbm, ip1_hbm, iq0_hbm, iq1_hbm, mp_hbm, mq_hbm,
             out_hbm, feat_v, ip0_v, ip1_v, iq0_v, iq1_v, mp_v, mq_v, st_v):
        wid = lax.axis_index("s") * nc + lax.axis_index("c")
        pltpu.sync_copy(feat_hbm.at[wid], feat_v)
        pltpu.sync_copy(ip0_hbm.at[wid], ip0_v)
        pltpu.sync_copy(ip1_hbm.at[wid], ip1_v)
        pltpu.sync_copy(iq0_hbm.at[wid], iq0_v)
        pltpu.sync_copy(iq1_hbm.at[wid], iq1_v)
        pltpu.sync_copy(mp_hbm.at[wid], mp_v)
        pltpu.sync_copy(mq_hbm.at[wid], mq_v)

        def gather_norm(idx):
            fs = [plsc.load_gather(feat_v, [idx + c * HW]) for c in range(C)]
            s = fs[0] * fs[0] + fs[1] * fs[1] + fs[2] * fs[2] + fs[3] * fs[3]
            r = _rsqrt_plus_eps_inv(s)
            return [f * r for f in fs]

        zero = jnp.zeros((L,), jnp.float32)
        pull_acc, pull_cnt, push_acc, push_cnt = zero, zero, zero, zero
        for j in range(K // L):
            sl = pl.ds(j * L, L)
            n0 = gather_norm(ip0_v[sl])
            n1 = gather_norm(ip1_v[sl])
            m = mp_v[sl]
            d2 = zero
            for a, b in zip(n0, n1):
                d = a - b
                d2 = d2 + d * d
            pull_acc = pull_acc + m * d2
            pull_cnt = pull_cnt + m

            p0 = gather_norm(iq0_v[sl])
            p1 = gather_norm(iq1_v[sl])
            mm = mq_v[sl]
            l1 = zero
            for a, b in zip(p0, p1):
                l1 = l1 + jnp.abs(a - b)
            push_acc = push_acc + mm * jnp.maximum(1.0 - l1, 0.0)
            push_cnt = push_cnt + mm

        st_v[...] = pull_acc
        pltpu.sync_copy(st_v, out_hbm.at[wid])
        st_v[...] = pull_cnt
        pltpu.sync_copy(st_v, out_hbm.at[B + wid])
        st_v[...] = push_acc
        pltpu.sync_copy(st_v, out_hbm.at[2 * B + wid])
        st_v[...] = push_cnt
        pltpu.sync_copy(st_v, out_hbm.at[3 * B + wid])

    return body(feat, ip0, ip1, iq0, iq1, mp, mq)


def _finalize_body(p_ref, o_ref):
    v = p_ref[...]
    ps = jnp.sum(v[0:1, :])
    pc = jnp.sum(v[1:2, :])
    qs = jnp.sum(v[2:3, :])
    qc = jnp.sum(v[3:4, :])
    loss = ps / (pc + 1e-4) + qs / (qc + 1e-4)
    o_ref[...] = jnp.full((1, 1), loss, jnp.float32)


def kernel(output, tag_pull, tag_push, mask_pull, mask_push):
    feat = output.reshape(B, C * HW)
    ip0 = tag_pull[:, :, 0].astype(jnp.int32)
    ip1 = tag_pull[:, :, 1].astype(jnp.int32)
    iq0 = tag_push[:, :, 0].astype(jnp.int32)
    iq1 = tag_push[:, :, 1].astype(jnp.int32)
    mp = mask_pull.astype(jnp.float32)
    mq = mask_push.astype(jnp.float32)
    partials = _sc_partials(feat, ip0, ip1, iq0, iq1, mp, mq)
    loss = pl.pallas_call(
        _finalize_body,
        out_shape=jax.ShapeDtypeStruct((1, 1), jnp.float32),
    )(partials.reshape(4, B * L))
    return loss[0, 0]


# speedup vs baseline: 1.6442x; 1.0587x over previous
import functools

import jax
import jax.numpy as jnp
from jax import lax
from jax.experimental import pallas as pl
from jax.experimental.pallas import tpu as pltpu
from jax.experimental.pallas import tpu_sc as plsc

B, C, H, W, K = 32, 4, 128, 128, 128
HW = H * W
L = 16


def _rsqrt_plus_eps_inv(s):
    xi = plsc.bitcast(s, jnp.int32)
    yi = jnp.int32(0x5F3759DF) - lax.shift_right_logical(xi, 1)
    y = plsc.bitcast(yi, jnp.float32)
    for _ in range(3):
        y = y * (1.5 - 0.5 * s * y * y)
    sqrt_s = s * y
    return 1.0 / (sqrt_s + 1e-10)


def _sc_distances(feat, tp, tq):
    info = plsc.get_sparse_core_info()
    nc = info.num_cores
    mesh = plsc.VectorSubcoreMesh(core_axis_name="c", subcore_axis_name="s")

    @functools.partial(
        pl.kernel,
        mesh=mesh,
        out_type=(
            jax.ShapeDtypeStruct((B, K), jnp.float32),
            jax.ShapeDtypeStruct((B, K), jnp.float32),
        ),
        compiler_params=pltpu.CompilerParams(needs_layout_passes=False),
        scratch_types=[
            pltpu.VMEM((C * HW,), jnp.float32),
            pltpu.VMEM((2 * K,), jnp.int32),
            pltpu.VMEM((2 * K,), jnp.int32),
            pltpu.VMEM((K,), jnp.float32),
            pltpu.VMEM((K,), jnp.float32),
            pltpu.SemaphoreType.DMA,
        ],
    )
    def body(feat_hbm, tp_hbm, tq_hbm, outp_hbm, outq_hbm,
             feat_v, tp_v, tq_v, d2_v, pt_v, sem):
        wid = lax.axis_index("s") * nc + lax.axis_index("c")
        cp = pltpu.async_copy(feat_hbm.at[wid], feat_v, sem)
        pltpu.sync_copy(tp_hbm.at[wid], tp_v)
        pltpu.sync_copy(tq_hbm.at[wid], tq_v)
        cp.wait()

        def gather_norm(idx):
            fs = [plsc.load_gather(feat_v, [idx + c * HW]) for c in range(C)]
            s = fs[0] * fs[0] + fs[1] * fs[1] + fs[2] * fs[2] + fs[3] * fs[3]
            r = _rsqrt_plus_eps_inv(s)
            return [f * r for f in fs]

        lanes2 = 2 * lax.iota(jnp.int32, L)
        for j in range(K // L):
            pos = 2 * L * j + lanes2
            n0 = gather_norm(plsc.load_gather(tp_v, [pos]))
            n1 = gather_norm(plsc.load_gather(tp_v, [pos + 1]))
            d2 = jnp.zeros((L,), jnp.float32)
            for a, b in zip(n0, n1):
                d = a - b
                d2 = d2 + d * d
            d2_v[pl.ds(j * L, L)] = d2

            p0 = gather_norm(plsc.load_gather(tq_v, [pos]))
            p1 = gather_norm(plsc.load_gather(tq_v, [pos + 1]))
            l1 = jnp.zeros((L,), jnp.float32)
            for a, b in zip(p0, p1):
                l1 = l1 + jnp.abs(a - b)
            pt_v[pl.ds(j * L, L)] = jnp.maximum(1.0 - l1, 0.0)

        pltpu.sync_copy(d2_v, outp_hbm.at[wid])
        pltpu.sync_copy(pt_v, outq_hbm.at[wid])

    return body(feat, tp, tq)


def _finalize_body(d2_ref, pt_ref, mp_ref, mq_ref, o_ref):
    mpf = mp_ref[...].astype(jnp.float32)
    mqf = mq_ref[...].astype(jnp.float32)
    ps = jnp.sum(d2_ref[...] * mpf)
    pc = jnp.sum(mpf)
    qs = jnp.sum(pt_ref[...] * mqf)
    qc = jnp.sum(mqf)
    loss = ps / (pc + 1e-4) + qs / (qc + 1e-4)
    o_ref[...] = jnp.full((1, 1), loss, jnp.float32)


def kernel(output, tag_pull, tag_push, mask_pull, mask_push):
    feat = output.reshape(B, C * HW)
    tp = tag_pull.reshape(B, 2 * K)
    tq = tag_push.reshape(B, 2 * K)
    d2, pt = _sc_distances(feat, tp, tq)
    loss = pl.pallas_call(
        _finalize_body,
        out_shape=jax.ShapeDtypeStruct((1, 1), jnp.float32),
    )(d2, pt, mask_pull, mask_push)
    return loss[0, 0]


# speedup vs baseline: 2.3338x; 1.4194x over previous
import functools

import jax
import jax.numpy as jnp
from jax import lax
from jax.experimental import pallas as pl
from jax.experimental.pallas import tpu as pltpu
from jax.experimental.pallas import tpu_sc as plsc

B, C, H, W, K = 32, 4, 128, 128, 128
HW = H * W
L = 16


def _rsqrt_plus_eps_inv(s):
    xi = plsc.bitcast(s, jnp.int32)
    yi = jnp.int32(0x5F3759DF) - lax.shift_right_logical(xi, 1)
    y = plsc.bitcast(yi, jnp.float32)
    for _ in range(3):
        y = y * (1.5 - 0.5 * s * y * y)
    sqrt_s = s * y
    return 1.0 / (sqrt_s + 1e-10)


def _sc_distances(feat, tp, tq):
    info = plsc.get_sparse_core_info()
    nc = info.num_cores
    mesh = plsc.VectorSubcoreMesh(core_axis_name="c", subcore_axis_name="s")

    @functools.partial(
        pl.kernel,
        mesh=mesh,
        out_type=(
            jax.ShapeDtypeStruct((B, K), jnp.float32),
            jax.ShapeDtypeStruct((B, K), jnp.float32),
        ),
        compiler_params=pltpu.CompilerParams(needs_layout_passes=False),
        scratch_types=[
            pltpu.VMEM((C * H, W), jnp.float32),
            pltpu.VMEM((2 * K,), jnp.int32),
            pltpu.VMEM((2 * K,), jnp.int32),
            pltpu.VMEM((K,), jnp.float32),
            pltpu.VMEM((K,), jnp.float32),
            pltpu.SemaphoreType.DMA,
        ],
    )
    def body(feat_hbm, tp_hbm, tq_hbm, outp_hbm, outq_hbm,
             feat_v, tp_v, tq_v, d2_v, pt_v, sem):
        wid = lax.axis_index("s") * nc + lax.axis_index("c")
        cp = pltpu.async_copy(feat_hbm.at[pl.ds(wid * C * H, C * H)], feat_v, sem)
        pltpu.sync_copy(tp_hbm.at[wid], tp_v)
        pltpu.sync_copy(tq_hbm.at[wid], tq_v)
        cp.wait()

        def gather_norm(idx):
            row = lax.shift_right_logical(idx, 7)
            col = lax.bitwise_and(idx, W - 1)
            fs = [plsc.load_gather(feat_v, [row + c * H, col]) for c in range(C)]
            s = fs[0] * fs[0] + fs[1] * fs[1] + fs[2] * fs[2] + fs[3] * fs[3]
            r = _rsqrt_plus_eps_inv(s)
            return [f * r for f in fs]

        lanes2 = 2 * lax.iota(jnp.int32, L)
        for j in range(K // L):
            pos = 2 * L * j + lanes2
            n0 = gather_norm(plsc.load_gather(tp_v, [pos]))
            n1 = gather_norm(plsc.load_gather(tp_v, [pos + 1]))
            d2 = jnp.zeros((L,), jnp.float32)
            for a, b in zip(n0, n1):
                d = a - b
                d2 = d2 + d * d
            d2_v[pl.ds(j * L, L)] = d2

            p0 = gather_norm(plsc.load_gather(tq_v, [pos]))
            p1 = gather_norm(plsc.load_gather(tq_v, [pos + 1]))
            l1 = jnp.zeros((L,), jnp.float32)
            for a, b in zip(p0, p1):
                l1 = l1 + jnp.abs(a - b)
            pt_v[pl.ds(j * L, L)] = jnp.maximum(1.0 - l1, 0.0)

        pltpu.sync_copy(d2_v, outp_hbm.at[wid])
        pltpu.sync_copy(pt_v, outq_hbm.at[wid])

    return body(feat, tp, tq)


def _finalize_body(d2_ref, pt_ref, mp_ref, mq_ref, o_ref):
    mpf = mp_ref[...].astype(jnp.float32)
    mqf = mq_ref[...].astype(jnp.float32)
    ps = jnp.sum(d2_ref[...] * mpf)
    pc = jnp.sum(mpf)
    qs = jnp.sum(pt_ref[...] * mqf)
    qc = jnp.sum(mqf)
    loss = ps / (pc + 1e-4) + qs / (qc + 1e-4)
    o_ref[...] = jnp.full((1, 1), loss, jnp.float32)


def kernel(output, tag_pull, tag_push, mask_pull, mask_push):
    feat = output.reshape(B * C * H, W)
    tp = tag_pull.reshape(B, 2 * K)
    tq = tag_push.reshape(B, 2 * K)
    d2, pt = _sc_distances(feat, tp, tq)
    loss = pl.pallas_call(
        _finalize_body,
        out_shape=jax.ShapeDtypeStruct((1, 1), jnp.float32),
    )(d2, pt, mask_pull, mask_push)
    return loss[0, 0]


# speedup vs baseline: 2.4877x; 1.0660x over previous
import functools

import jax
import jax.numpy as jnp
from jax import lax
from jax.experimental import pallas as pl
from jax.experimental.pallas import tpu as pltpu
from jax.experimental.pallas import tpu_sc as plsc

B, C, H, W, K = 32, 4, 128, 128, 128
HW = H * W
L = 16


def _rsqrt_plus_eps_inv(s):
    xi = plsc.bitcast(s, jnp.int32)
    yi = jnp.int32(0x5F3759DF) - lax.shift_right_logical(xi, 1)
    y = plsc.bitcast(yi, jnp.float32)
    for _ in range(3):
        y = y * (1.5 - 0.5 * s * y * y)
    sqrt_s = s * y
    return 1.0 / (sqrt_s + 1e-10)


def _sc_distances(feat, tp, tq):
    info = plsc.get_sparse_core_info()
    nc = info.num_cores
    mesh = plsc.VectorSubcoreMesh(core_axis_name="c", subcore_axis_name="s")

    @functools.partial(
        pl.kernel,
        mesh=mesh,
        out_type=(
            jax.ShapeDtypeStruct((B, K), jnp.float32),
            jax.ShapeDtypeStruct((B, K), jnp.float32),
        ),
        compiler_params=pltpu.CompilerParams(needs_layout_passes=False),
        scratch_types=[
            pltpu.VMEM((C * H, W), jnp.float32),
            pltpu.VMEM((2, K), jnp.int32),
            pltpu.VMEM((2, K), jnp.int32),
            pltpu.VMEM((K,), jnp.float32),
            pltpu.VMEM((K,), jnp.float32),
            pltpu.SemaphoreType.DMA,
        ],
    )
    def body(feat_hbm, tp_hbm, tq_hbm, outp_hbm, outq_hbm,
             feat_v, tp_v, tq_v, d2_v, pt_v, sem):
        wid = lax.axis_index("s") * nc + lax.axis_index("c")
        cp = pltpu.async_copy(feat_hbm.at[pl.ds(wid * C * H, C * H)], feat_v, sem)
        pltpu.sync_copy(tp_hbm.at[pl.ds(2 * wid, 2)], tp_v)
        pltpu.sync_copy(tq_hbm.at[pl.ds(2 * wid, 2)], tq_v)
        cp.wait()

        def gather_norm(idx):
            row = lax.shift_right_logical(idx, 7)
            col = lax.bitwise_and(idx, W - 1)
            fs = [plsc.load_gather(feat_v, [row + c * H, col]) for c in range(C)]
            s = fs[0] * fs[0] + fs[1] * fs[1] + fs[2] * fs[2] + fs[3] * fs[3]
            r = _rsqrt_plus_eps_inv(s)
            return [f * r for f in fs]

        def chunk(j, _):
            sl = pl.ds(j * L, L)
            n0 = gather_norm(tp_v[0, sl])
            n1 = gather_norm(tp_v[1, sl])
            d2 = jnp.zeros((L,), jnp.float32)
            for a, b in zip(n0, n1):
                d = a - b
                d2 = d2 + d * d
            d2_v[sl] = d2

            p0 = gather_norm(tq_v[0, sl])
            p1 = gather_norm(tq_v[1, sl])
            l1 = jnp.zeros((L,), jnp.float32)
            for a, b in zip(p0, p1):
                l1 = l1 + jnp.abs(a - b)
            pt_v[sl] = jnp.maximum(1.0 - l1, 0.0)
            return 0

        lax.fori_loop(0, K // L, chunk, 0, unroll=False)

        pltpu.sync_copy(d2_v, outp_hbm.at[wid])
        pltpu.sync_copy(pt_v, outq_hbm.at[wid])

    return body(feat, tp, tq)


def _finalize_body(d2_ref, pt_ref, mp_ref, mq_ref, o_ref):
    mpf = mp_ref[...].astype(jnp.float32)
    mqf = mq_ref[...].astype(jnp.float32)
    ps = jnp.sum(d2_ref[...] * mpf)
    pc = jnp.sum(mpf)
    qs = jnp.sum(pt_ref[...] * mqf)
    qc = jnp.sum(mqf)
    loss = ps / (pc + 1e-4) + qs / (qc + 1e-4)
    o_ref[...] = jnp.full((1, 1), loss, jnp.float32)


def kernel(output, tag_pull, tag_push, mask_pull, mask_push):
    feat = output.reshape(B * C * H, W)
    tp = tag_pull.transpose(0, 2, 1).reshape(2 * B, K)
    tq = tag_push.transpose(0, 2, 1).reshape(2 * B, K)
    d2, pt = _sc_distances(feat, tp, tq)
    loss = pl.pallas_call(
        _finalize_body,
        out_shape=jax.ShapeDtypeStruct((1, 1), jnp.float32),
    )(d2, pt, mask_pull, mask_push)
    return loss[0, 0]
